# baseline (speedup 1.0000x reference)
import functools

import jax
import jax.numpy as jnp
from jax import lax
from jax.experimental import pallas as pl
from jax.experimental.pallas import tpu as pltpu
from jax.experimental.pallas import tpu_sc as plsc

NC = 2
NS = 16
NW = NC * NS

GRP = 128
NGRP = 13
CHUNK = GRP * NGRP


def _make_sc_gather(n_chunks_total, n_chunks_per_w, flat_rows, dim):
    mesh = plsc.VectorSubcoreMesh(core_axis_name="c", subcore_axis_name="s")

    @functools.partial(
        pl.kernel,
        mesh=mesh,
        compiler_params=pltpu.CompilerParams(use_tc_tiling_on_sc=False),
        out_type=jax.ShapeDtypeStruct((n_chunks_total, NGRP, GRP, dim),
                                      jnp.float32),
        scratch_types=[
            pltpu.VMEM((NGRP, GRP), jnp.int32),
            pltpu.VMEM((NGRP, GRP), jnp.int32),
            pltpu.VMEM((NGRP, GRP, dim), jnp.float32),
            pltpu.SemaphoreType.DMA,
        ],
    )
    def sc_gather(idx_hbm, tab_hbm, offs_hbm, out_hbm,
                  offs_v, gidx_v, rows_v, sem):
        wid = lax.axis_index("s") * NC + lax.axis_index("c")
        pltpu.sync_copy(offs_hbm, offs_v)

        def chunk_body(c, carry):
            cid = wid * n_chunks_per_w + c
            pltpu.sync_copy(idx_hbm.at[cid], gidx_v)

            def add_body(j, carry2):
                for k in range(GRP // 16):
                    sl = pl.ds(k * 16, 16)
                    gidx_v[j, sl] = gidx_v[j, sl] + offs_v[j, sl]
                return carry2

            lax.fori_loop(0, NGRP, add_body, 0, unroll=True)

            cps = [pltpu.async_copy(tab_hbm.at[gidx_v.at[j]],
                                    rows_v.at[j], sem)
                   for j in range(NGRP)]
            for cp in cps:
                cp.wait()
            pltpu.sync_copy(rows_v, out_hbm.at[cid])
            return carry

        lax.fori_loop(0, n_chunks_per_w, chunk_body, 0)

    return sc_gather


def kernel(indices, tables):
    batch, n_fields = indices.shape
    _, vocab, dim = tables.shape
    rows = batch * n_fields
    assert rows % (NW * CHUNK) == 0
    n_chunks_per_w = rows // (NW * CHUNK)
    n_chunks_total = rows // CHUNK

    idx3 = indices.reshape(n_chunks_total, NGRP, GRP)
    tab2 = tables.reshape(n_fields * vocab, dim)
    offs = ((jnp.arange(CHUNK, dtype=jnp.int32) % n_fields) *
            jnp.int32(vocab)).reshape(NGRP, GRP)

    out = _make_sc_gather(n_chunks_total, n_chunks_per_w, rows, dim)(
        idx3, tab2, offs)
    return out.reshape(batch, n_fields * dim)


# speedup vs baseline: 1.0076x; 1.0076x over previous
import functools

import jax
import jax.numpy as jnp
from jax import lax
from jax.experimental import pallas as pl
from jax.experimental.pallas import tpu as pltpu
from jax.experimental.pallas import tpu_sc as plsc

NC = 2
NS = 16
NW = NC * NS

GRP = 128
NGRP = 13
CHUNK = GRP * NGRP


def _make_sc_gather(n_chunks_total, n_chunks_per_w, dim):
    mesh = plsc.VectorSubcoreMesh(core_axis_name="c", subcore_axis_name="s")
    ncw = n_chunks_per_w

    @functools.partial(
        pl.kernel,
        mesh=mesh,
        compiler_params=pltpu.CompilerParams(use_tc_tiling_on_sc=False),
        out_type=jax.ShapeDtypeStruct((n_chunks_total, NGRP, GRP, dim),
                                      jnp.float32),
        scratch_types=[
            pltpu.VMEM((NGRP, GRP), jnp.int32),
            pltpu.VMEM((ncw, NGRP, GRP), jnp.int32),
            pltpu.VMEM((2, NGRP, GRP, dim), jnp.float32),
            pltpu.SemaphoreType.DMA,
            pltpu.SemaphoreType.DMA,
            pltpu.SemaphoreType.DMA,
            pltpu.SemaphoreType.DMA,
        ],
    )
    def sc_gather(idx_hbm, tab_hbm, offs_hbm, out_hbm,
                  offs_v, gidx_v, rows_v, g0, g1, s0, s1):
        wid = lax.axis_index("s") * NC + lax.axis_index("c")
        base = wid * ncw
        pltpu.sync_copy(offs_hbm, offs_v)
        pltpu.sync_copy(idx_hbm.at[pl.ds(base, ncw)], gidx_v)

        gsem = (g0, g1)
        ssem = (s0, s1)

        def add_pass(c):
            def add_body(j, carry):
                for k in range(GRP // 16):
                    sl = pl.ds(k * 16, 16)
                    gidx_v[c, j, sl] = gidx_v[c, j, sl] + offs_v[j, sl]
                return carry
            lax.fori_loop(0, NGRP, add_body, 0, unroll=True)

        def fire(c):
            b = c % 2
            return [pltpu.async_copy(tab_hbm.at[gidx_v.at[c, j]],
                                     rows_v.at[b, j], gsem[b])
                    for j in range(NGRP)]

        def store(c):
            b = c % 2
            return pltpu.async_copy(rows_v.at[b], out_hbm.at[base + c],
                                    ssem[b])

        gh = {}
        sh = {}
        for c in range(ncw):
            if c >= 2:
                sh[c - 2].wait()
            add_pass(c)
            gh[c] = fire(c)
            if c >= 1:
                for h in gh.pop(c - 1):
                    h.wait()
                sh[c - 1] = store(c - 1)
        for h in gh.pop(ncw - 1):
            h.wait()
        sh[ncw - 1] = store(ncw - 1)
        sh[ncw - 2].wait()
        sh[ncw - 1].wait()

    return sc_gather


def kernel(indices, tables):
    batch, n_fields = indices.shape
    _, vocab, dim = tables.shape
    rows = batch * n_fields
    assert rows % (NW * CHUNK) == 0
    n_chunks_per_w = rows // (NW * CHUNK)
    n_chunks_total = rows // CHUNK

    idx3 = indices.reshape(n_chunks_total, NGRP, GRP)
    tab2 = tables.reshape(n_fields * vocab, dim)
    offs = ((jnp.arange(CHUNK, dtype=jnp.int32) % n_fields) *
            jnp.int32(vocab)).reshape(NGRP, GRP)

    out = _make_sc_gather(n_chunks_total, n_chunks_per_w, dim)(
        idx3, tab2, offs)
    return out.reshape(batch, n_fields * dim)
